# baseline (speedup 1.0000x reference)
import functools

import jax
import jax.numpy as jnp
from jax import lax
from jax.experimental import pallas as pl
from jax.experimental.pallas import tpu as pltpu
from jax.experimental.pallas import tpu_sc as plsc

N_PROTO = 256
IN_DIM = 1024
N_PATCH = 32768
BM = 4096
GRID = N_PATCH // BM
_LANES = 16


def _proto_count_kernel(x_ref, pt_ref, out_ref):
    pt = pt_ref[...]
    ones_k = jnp.full((1, IN_DIM), 0.5, jnp.float32)
    hpsq = jax.lax.dot_general(
        ones_k, pt * pt,
        (((1,), (1,)), ((), ())),
        preferred_element_type=jnp.float32,
    )
    dots = jax.lax.dot_general(
        x_ref[...], pt,
        (((1,), (1,)), ((), ())),
        preferred_element_type=jnp.float32,
        precision=jax.lax.Precision.DEFAULT,
    )
    m = dots - hpsq
    rowmax = jnp.max(m, axis=1, keepdims=True)
    onehot = jnp.where(m == rowmax, 1.0, 0.0)
    ones_m = jnp.ones((1, BM), jnp.float32)
    out_ref[0] = jax.lax.dot_general(
        ones_m, onehot,
        (((1,), (0,)), ((), ())),
        preferred_element_type=jnp.float32,
    )


def _partial_counts(x, prototypes):
    return pl.pallas_call(
        _proto_count_kernel,
        grid=(GRID,),
        in_specs=[
            pl.BlockSpec((BM, IN_DIM), lambda i: (i, 0)),
            pl.BlockSpec((N_PROTO, IN_DIM), lambda i: (0, 0)),
        ],
        out_specs=pl.BlockSpec((1, 1, N_PROTO), lambda i: (i, 0, 0)),
        out_shape=jax.ShapeDtypeStruct((GRID, 1, N_PROTO), jnp.float32),
        compiler_params=pltpu.CompilerParams(
            dimension_semantics=("arbitrary",),
        ),
    )(x, prototypes)


def _sqrt_sc(s):
    y = jnp.full((_LANES,), 6000.0, jnp.float32)
    for _ in range(8):
        y = 0.5 * (y + s / y)
    return y


_SC_MESH = plsc.VectorSubcoreMesh(core_axis_name="c", subcore_axis_name="s")


@functools.partial(
    pl.kernel,
    out_type=jax.ShapeDtypeStruct((1, N_PROTO), jnp.float32),
    mesh=_SC_MESH,
    scratch_types=[
        pltpu.VMEM((GRID, 1, N_PROTO), jnp.float32),
        pltpu.VMEM((1, N_PROTO), jnp.float32),
    ],
)
def _sc_combine(parts_hbm, out_hbm, parts_v, nrm_v):
    cid = lax.axis_index("c")
    sid = lax.axis_index("s")

    @pl.when((cid == 0) & (sid == 0))
    def _():
        pltpu.sync_copy(parts_hbm, parts_v)
        ssv = jnp.zeros((_LANES,), jnp.float32)
        for chunk in range(N_PROTO // _LANES):
            sl = pl.ds(chunk * _LANES, _LANES)
            acc = parts_v[0, 0, sl]
            for r in range(1, GRID):
                acc = acc + parts_v[r, 0, sl]
            nrm_v[0, sl] = acc
            ssv = ssv + acc * acc
        ss = ssv[0]
        for k in range(1, _LANES):
            ss = ss + ssv[k]
        total = jnp.broadcast_to(ss, (_LANES,))
        nrm = _sqrt_sc(total)
        for chunk in range(N_PROTO // _LANES):
            sl = pl.ds(chunk * _LANES, _LANES)
            nrm_v[0, sl] = nrm_v[0, sl] / nrm
        pltpu.sync_copy(nrm_v, out_hbm)


@jax.jit
def kernel(x, prototypes):
    parts = _partial_counts(x, prototypes)
    return _sc_combine(parts)


# speedup vs baseline: 1.3473x; 1.3473x over previous
import functools

import jax
import jax.numpy as jnp
from jax import lax
from jax.experimental import pallas as pl
from jax.experimental.pallas import tpu as pltpu
from jax.experimental.pallas import tpu_sc as plsc

N_PROTO = 256
IN_DIM = 1024
N_PATCH = 32768
BM = 4096
GRID = N_PATCH // BM
_LANES = 16


def _proto_count_kernel(x_ref, pt_ref, out_ref):
    pt = pt_ref[...]
    ones_k = jnp.full((1, IN_DIM), 0.5, jnp.float32)
    hpsq = jax.lax.dot_general(
        ones_k, pt * pt,
        (((1,), (1,)), ((), ())),
        preferred_element_type=jnp.float32,
    )
    dots = jax.lax.dot_general(
        x_ref[...], pt,
        (((1,), (1,)), ((), ())),
        preferred_element_type=jnp.float32,
        precision=jax.lax.Precision.DEFAULT,
    )
    m = dots - hpsq
    rowmax = jnp.max(m, axis=1, keepdims=True)
    onehot = jnp.where(m == rowmax, 1.0, 0.0)
    ones_m = jnp.ones((1, BM), jnp.float32)
    out_ref[0] = jax.lax.dot_general(
        ones_m, onehot,
        (((1,), (0,)), ((), ())),
        preferred_element_type=jnp.float32,
    )


def _partial_counts(x, prototypes):
    return pl.pallas_call(
        _proto_count_kernel,
        grid=(GRID,),
        in_specs=[
            pl.BlockSpec((BM, IN_DIM), lambda i: (i, 0)),
            pl.BlockSpec((N_PROTO, IN_DIM), lambda i: (0, 0)),
        ],
        out_specs=pl.BlockSpec((1, 1, N_PROTO), lambda i: (i, 0, 0)),
        out_shape=jax.ShapeDtypeStruct((GRID, 1, N_PROTO), jnp.float32),
        compiler_params=pltpu.CompilerParams(
            dimension_semantics=("parallel",),
        ),
    )(x, prototypes)


def _combine_kernel(parts_ref, out_ref):
    c = jnp.sum(parts_ref[...], axis=0)
    out_ref[...] = c * jax.lax.rsqrt(jnp.sum(c * c))


def _combine_tc(parts):
    return pl.pallas_call(
        _combine_kernel,
        out_shape=jax.ShapeDtypeStruct((1, N_PROTO), jnp.float32),
    )(parts)


def _sqrt_sc(s):
    y = jnp.full((_LANES,), 6000.0, jnp.float32)
    for _ in range(8):
        y = 0.5 * (y + s / y)
    return y


_SC_MESH = plsc.VectorSubcoreMesh(core_axis_name="c", subcore_axis_name="s")


@functools.partial(
    pl.kernel,
    out_type=jax.ShapeDtypeStruct((1, N_PROTO), jnp.float32),
    mesh=_SC_MESH,
    scratch_types=[
        pltpu.VMEM((GRID, 1, N_PROTO), jnp.float32),
        pltpu.VMEM((1, N_PROTO), jnp.float32),
    ],
)
def _sc_combine(parts_hbm, out_hbm, parts_v, nrm_v):
    cid = lax.axis_index("c")
    sid = lax.axis_index("s")

    @pl.when((cid == 0) & (sid == 0))
    def _():
        pltpu.sync_copy(parts_hbm, parts_v)
        ssv = jnp.zeros((_LANES,), jnp.float32)
        for chunk in range(N_PROTO // _LANES):
            sl = pl.ds(chunk * _LANES, _LANES)
            acc = parts_v[0, 0, sl]
            for r in range(1, GRID):
                acc = acc + parts_v[r, 0, sl]
            nrm_v[0, sl] = acc
            ssv = ssv + acc * acc
        ss = ssv[0]
        for k in range(1, _LANES):
            ss = ss + ssv[k]
        total = jnp.broadcast_to(ss, (_LANES,))
        nrm = _sqrt_sc(total)
        for chunk in range(N_PROTO // _LANES):
            sl = pl.ds(chunk * _LANES, _LANES)
            nrm_v[0, sl] = nrm_v[0, sl] / nrm
        pltpu.sync_copy(nrm_v, out_hbm)


@jax.jit
def kernel(x, prototypes):
    parts = _partial_counts(x, prototypes)
    return _combine_tc(parts)


# speedup vs baseline: 1.3823x; 1.0260x over previous
import functools

import jax
import jax.numpy as jnp
from jax.experimental import pallas as pl
from jax.experimental.pallas import tpu as pltpu

N_PROTO = 256
IN_DIM = 1024
N_PATCH = 32768
BM = 4096


def _proto_count_kernel(x_ref, pt_ref, out_ref):
    i = pl.program_id(0)

    @pl.when(i == 0)
    def _init():
        out_ref[...] = jnp.zeros_like(out_ref)

    pt = pt_ref[...]
    ones_k = jnp.full((1, IN_DIM), 0.5, jnp.float32)
    hpsq = jax.lax.dot_general(
        ones_k, pt * pt,
        (((1,), (1,)), ((), ())),
        preferred_element_type=jnp.float32,
    )
    dots = jax.lax.dot_general(
        x_ref[...], pt,
        (((1,), (1,)), ((), ())),
        preferred_element_type=jnp.float32,
        precision=jax.lax.Precision.DEFAULT,
    )
    m = dots - hpsq
    rowmax = jnp.max(m, axis=1, keepdims=True)
    onehot = (m >= rowmax).astype(jnp.float32)
    ones_m = jnp.ones((1, BM), jnp.float32)
    out_ref[...] += jax.lax.dot_general(
        ones_m, onehot,
        (((1,), (0,)), ((), ())),
        preferred_element_type=jnp.float32,
    )

    @pl.when(i == pl.num_programs(0) - 1)
    def _finish():
        c = out_ref[...]
        out_ref[...] = c * jax.lax.rsqrt(jnp.sum(c * c))


@jax.jit
def kernel(x, prototypes):
    return pl.pallas_call(
        _proto_count_kernel,
        grid=(N_PATCH // BM,),
        in_specs=[
            pl.BlockSpec((BM, IN_DIM), lambda i: (i, 0)),
            pl.BlockSpec((N_PROTO, IN_DIM), lambda i: (0, 0)),
        ],
        out_specs=pl.BlockSpec((1, N_PROTO), lambda i: (0, 0)),
        out_shape=jax.ShapeDtypeStruct((1, N_PROTO), jnp.float32),
        compiler_params=pltpu.CompilerParams(
            dimension_semantics=("arbitrary",),
        ),
    )(x, prototypes)
